# Initial kernel scaffold; baseline (speedup 1.0000x reference)
import jax
import jax.numpy as jnp
from jax.experimental import pallas as pl


def kernel(vertices, faces, face_probs):
    raise NotImplementedError("write your pallas kernel here")


# speedup vs baseline: 150.8328x; 150.8328x over previous
import functools

import jax
import jax.numpy as jnp
from jax import lax
from jax.experimental import pallas as pl
from jax.experimental.pallas import tpu as pltpu
from jax.experimental.pallas import tpu_sc as plsc

V = 4096
F = 8192

NC = 2
NS = 16
NW = NC * NS
L = 16
CHUNK = F // NW
G = CHUNK // L

KROWS = 56

TI = 512
TB = 512


def _pre_body(v_hbm, f_hbm, p_hbm, out_hbm, v_v, f_v, p_v, d_v):
    wid = lax.axis_index("s") * NC + lax.axis_index("c")
    base = wid * CHUNK
    pltpu.sync_copy(v_hbm, v_v)
    pltpu.sync_copy(f_hbm.at[pl.ds(base * 3, CHUNK * 3)], f_v)
    pltpu.sync_copy(p_hbm.at[pl.ds(base, CHUNK)], p_v)

    zf = jnp.zeros((L,), jnp.float32)
    onef = jnp.full((L,), 1.0, jnp.float32)
    negf = jnp.full((L,), -1.0, jnp.float32)

    for g in range(G):
        sl = pl.ds(g * L, L)
        fid3 = (lax.iota(jnp.int32, L) + g * L) * 3
        i0 = plsc.load_gather(f_v, [fid3]) * 3
        i1 = plsc.load_gather(f_v, [fid3 + 1]) * 3
        i2 = plsc.load_gather(f_v, [fid3 + 2]) * 3
        p0x = plsc.load_gather(v_v, [i0])
        p0y = plsc.load_gather(v_v, [i0 + 1])
        p0z = plsc.load_gather(v_v, [i0 + 2])
        p1x = plsc.load_gather(v_v, [i1])
        p1y = plsc.load_gather(v_v, [i1 + 1])
        p1z = plsc.load_gather(v_v, [i1 + 2])
        p2x = plsc.load_gather(v_v, [i2])
        p2y = plsc.load_gather(v_v, [i2 + 1])
        p2z = plsc.load_gather(v_v, [i2 + 2])

        e1x = p1x - p0x
        e1y = p1y - p0y
        e1z = p1z - p0z
        e2x = p2x - p0x
        e2y = p2y - p0y
        e2z = p2z - p0z
        nx = e1y * e2z - e1z * e2y
        ny = e1z * e2x - e1x * e2z
        nz = e1x * e2y - e1y * e2x
        ns = nx + ny + nz

        cx = (p0x + p1x + p2x) / 3.0
        cy = (p0y + p1y + p2y) / 3.0
        cz = (p0z + p1z + p2z) / 3.0
        sq = cx * cx + cy * cy + cz * cz
        pr = p_v[sl]

        d_v[0, sl] = nx
        d_v[1, sl] = ny
        d_v[2, sl] = nz
        d_v[3, sl] = p0x * ns
        d_v[4, sl] = p0y * ns
        d_v[5, sl] = p0z * ns
        d_v[8, sl] = p0x
        d_v[9, sl] = p1x
        d_v[10, sl] = p2x
        d_v[11, sl] = negf
        d_v[16, sl] = p0y
        d_v[17, sl] = p1y
        d_v[18, sl] = p2y
        d_v[20, sl] = negf
        d_v[24, sl] = p0z
        d_v[25, sl] = p1z
        d_v[26, sl] = p2z
        d_v[29, sl] = negf
        d_v[32, sl] = cx
        d_v[33, sl] = cy
        d_v[34, sl] = cz
        d_v[35, sl] = sq
        d_v[36, sl] = onef
        d_v[40, sl] = -2.0 * cx
        d_v[41, sl] = -2.0 * cy
        d_v[42, sl] = -2.0 * cz
        d_v[43, sl] = onef
        d_v[44, sl] = sq - 1.0
        d_v[48, sl] = pr
        for r in (6, 7, 12, 13, 14, 15, 19, 21, 22, 23,
                  27, 28, 30, 31, 37, 38, 39, 45, 46, 47):
            d_v[r, sl] = zf

    pltpu.sync_copy(d_v, out_hbm.at[:, pl.ds(base, CHUNK)])


@functools.cache
def _make_pre():
    return pl.kernel(
        _pre_body,
        out_type=jax.ShapeDtypeStruct((KROWS, F), jnp.float32),
        mesh=plsc.VectorSubcoreMesh(
            core_axis_name="c", subcore_axis_name="s",
            num_cores=NC, num_subcores=NS),
        scratch_types=[
            pltpu.VMEM((V * 3,), jnp.float32),
            pltpu.VMEM((CHUNK * 3,), jnp.int32),
            pltpu.VMEM((CHUNK,), jnp.float32),
            pltpu.VMEM((KROWS, CHUNK), jnp.float32),
        ],
        compiler_params=pltpu.CompilerParams(needs_layout_passes=False),
    )


def _pair_body(a_ref, b_ref, o_ref):
    a = a_ref[...]
    b = b_ref[...]
    dn = (((0,), (0,)), ((), ()))
    hp = lax.Precision.HIGHEST
    s0 = lax.dot_general(a[0:8], b[8:16], dn, precision=hp,
                         preferred_element_type=jnp.float32)
    s1 = lax.dot_general(a[0:8], b[16:24], dn, precision=hp,
                         preferred_element_type=jnp.float32)
    s2 = lax.dot_general(a[0:8], b[24:32], dn, precision=hp,
                         preferred_element_type=jnp.float32)
    h = lax.dot_general(a[32:40], b[40:48], dn, precision=hp,
                        preferred_element_type=jnp.float32)
    inter = ((s0 * s2) < 0.0) | ((s1 * s0) < 0.0) | ((s2 * s1) < 0.0)
    ii = pl.program_id(0)
    bb = pl.program_id(1)
    ri = ii * TI + lax.broadcasted_iota(jnp.int32, (TI, TB), 0)
    ci = bb * TB + lax.broadcasted_iota(jnp.int32, (TI, TB), 1)
    w = (inter & (h < 0.0) & (ri != ci)).astype(jnp.float32)
    pr = a[48:49]
    row = lax.dot_general(pr, w, (((1,), (0,)), ((), ())), precision=hp,
                          preferred_element_type=jnp.float32)
    val = jnp.sum(row)

    @pl.when((ii == 0) & (bb == 0))
    def _init():
        o_ref[0, 0] = 0.0

    o_ref[0, 0] += val


_pair = pl.pallas_call(
    _pair_body,
    grid=(F // TI, F // TB),
    in_specs=[
        pl.BlockSpec((KROWS, TI), lambda i, b: (0, i)),
        pl.BlockSpec((KROWS, TB), lambda i, b: (0, b)),
    ],
    out_specs=pl.BlockSpec(memory_space=pltpu.SMEM),
    out_shape=jax.ShapeDtypeStruct((1, 1), jnp.float32),
)


def kernel(vertices, faces, face_probs):
    d = _make_pre()(vertices.reshape(-1), faces.reshape(-1), face_probs)
    tot = _pair(d, d)
    return tot[0, 0] / jnp.float32(F)


# speedup vs baseline: 448.2622x; 2.9719x over previous
import functools

import jax
import jax.numpy as jnp
from jax import lax
from jax.experimental import pallas as pl
from jax.experimental.pallas import tpu as pltpu
from jax.experimental.pallas import tpu_sc as plsc

V = 4096
F = 8192

NC = 2
NS = 16
NW = NC * NS
L = 16
CHUNK = F // NW
G = CHUNK // L

KROWS = 56

TI = 512
TB = 512


def _pre_body(v_hbm, f_hbm, p_hbm, out_hbm, v_v, f_v, p_v, d_v):
    wid = lax.axis_index("s") * NC + lax.axis_index("c")
    base = wid * CHUNK
    pltpu.sync_copy(v_hbm, v_v)
    pltpu.sync_copy(f_hbm.at[pl.ds(base * 3, CHUNK * 3)], f_v)
    pltpu.sync_copy(p_hbm.at[pl.ds(base, CHUNK)], p_v)

    zf = jnp.zeros((L,), jnp.float32)
    onef = jnp.full((L,), 1.0, jnp.float32)
    negf = jnp.full((L,), -1.0, jnp.float32)

    for g in range(G):
        sl = pl.ds(g * L, L)
        fid3 = (lax.iota(jnp.int32, L) + g * L) * 3
        i0 = plsc.load_gather(f_v, [fid3]) * 3
        i1 = plsc.load_gather(f_v, [fid3 + 1]) * 3
        i2 = plsc.load_gather(f_v, [fid3 + 2]) * 3
        p0x = plsc.load_gather(v_v, [i0])
        p0y = plsc.load_gather(v_v, [i0 + 1])
        p0z = plsc.load_gather(v_v, [i0 + 2])
        p1x = plsc.load_gather(v_v, [i1])
        p1y = plsc.load_gather(v_v, [i1 + 1])
        p1z = plsc.load_gather(v_v, [i1 + 2])
        p2x = plsc.load_gather(v_v, [i2])
        p2y = plsc.load_gather(v_v, [i2 + 1])
        p2z = plsc.load_gather(v_v, [i2 + 2])

        e1x = p1x - p0x
        e1y = p1y - p0y
        e1z = p1z - p0z
        e2x = p2x - p0x
        e2y = p2y - p0y
        e2z = p2z - p0z
        nx = e1y * e2z - e1z * e2y
        ny = e1z * e2x - e1x * e2z
        nz = e1x * e2y - e1y * e2x
        ns = nx + ny + nz

        cx = (p0x + p1x + p2x) / 3.0
        cy = (p0y + p1y + p2y) / 3.0
        cz = (p0z + p1z + p2z) / 3.0
        sq = cx * cx + cy * cy + cz * cz
        pr = p_v[sl]

        d_v[0, sl] = nx
        d_v[1, sl] = ny
        d_v[2, sl] = nz
        d_v[3, sl] = p0x * ns
        d_v[4, sl] = p0y * ns
        d_v[5, sl] = p0z * ns
        d_v[8, sl] = p0x
        d_v[9, sl] = p1x
        d_v[10, sl] = p2x
        d_v[11, sl] = negf
        d_v[16, sl] = p0y
        d_v[17, sl] = p1y
        d_v[18, sl] = p2y
        d_v[20, sl] = negf
        d_v[24, sl] = p0z
        d_v[25, sl] = p1z
        d_v[26, sl] = p2z
        d_v[29, sl] = negf
        d_v[32, sl] = cx
        d_v[33, sl] = cy
        d_v[34, sl] = cz
        d_v[35, sl] = sq
        d_v[36, sl] = onef
        d_v[40, sl] = -2.0 * cx
        d_v[41, sl] = -2.0 * cy
        d_v[42, sl] = -2.0 * cz
        d_v[43, sl] = onef
        d_v[44, sl] = sq - 1.0
        d_v[48, sl] = pr
        for r in (6, 7, 12, 13, 14, 15, 19, 21, 22, 23,
                  27, 28, 30, 31, 37, 38, 39, 45, 46, 47):
            d_v[r, sl] = zf

    pltpu.sync_copy(d_v, out_hbm.at[:, pl.ds(base, CHUNK)])


@functools.cache
def _make_pre():
    return pl.kernel(
        _pre_body,
        out_type=jax.ShapeDtypeStruct((KROWS, F), jnp.float32),
        mesh=plsc.VectorSubcoreMesh(
            core_axis_name="c", subcore_axis_name="s",
            num_cores=NC, num_subcores=NS),
        scratch_types=[
            pltpu.VMEM((V * 3,), jnp.float32),
            pltpu.VMEM((CHUNK * 3,), jnp.int32),
            pltpu.VMEM((CHUNK,), jnp.float32),
            pltpu.VMEM((KROWS, CHUNK), jnp.float32),
        ],
        compiler_params=pltpu.CompilerParams(needs_layout_passes=False),
    )


def _pair_body(a_ref, b_ref, o_ref):
    a = a_ref[...]
    b = b_ref[...]
    dn = (((0,), (0,)), ((), ()))
    hp = None
    s0 = lax.dot_general(a[0:8], b[8:16], dn, precision=hp,
                         preferred_element_type=jnp.float32)
    s1 = lax.dot_general(a[0:8], b[16:24], dn, precision=hp,
                         preferred_element_type=jnp.float32)
    s2 = lax.dot_general(a[0:8], b[24:32], dn, precision=hp,
                         preferred_element_type=jnp.float32)
    h = lax.dot_general(a[32:40], b[40:48], dn, precision=hp,
                        preferred_element_type=jnp.float32)
    inter = ((s0 * s2) < 0.0) | ((s1 * s0) < 0.0) | ((s2 * s1) < 0.0)
    ii = pl.program_id(0)
    bb = pl.program_id(1)
    ri = ii * TI + lax.broadcasted_iota(jnp.int32, (TI, 1), 0)
    ci = bb * TB + lax.broadcasted_iota(jnp.int32, (1, TB), 1)
    w = (inter & (h < 0.0) & (ri != ci)).astype(jnp.float32)
    pr = a[48:49]
    row = lax.dot_general(pr, w, (((1,), (0,)), ((), ())), precision=hp,
                          preferred_element_type=jnp.float32)
    val = jnp.sum(row)

    @pl.when((ii == 0) & (bb == 0))
    def _init():
        o_ref[0, 0] = 0.0

    o_ref[0, 0] += val


_pair = pl.pallas_call(
    _pair_body,
    grid=(F // TI, F // TB),
    in_specs=[
        pl.BlockSpec((KROWS, TI), lambda i, b: (0, i)),
        pl.BlockSpec((KROWS, TB), lambda i, b: (0, b)),
    ],
    out_specs=pl.BlockSpec(memory_space=pltpu.SMEM),
    out_shape=jax.ShapeDtypeStruct((1, 1), jnp.float32),
    compiler_params=pltpu.CompilerParams(fuse_transposed_lhs_in_matmul=True),
)


def kernel(vertices, faces, face_probs):
    d = _make_pre()(vertices.reshape(-1), faces.reshape(-1), face_probs)
    tot = _pair(d, d)
    return tot[0, 0] / jnp.float32(F)


# speedup vs baseline: 647.6806x; 1.4449x over previous
import functools

import jax
import jax.numpy as jnp
from jax import lax
from jax.experimental import pallas as pl
from jax.experimental.pallas import tpu as pltpu
from jax.experimental.pallas import tpu_sc as plsc

V = 4096
F = 8192

NC = 2
NS = 16
NW = NC * NS
L = 16
CHUNK = F // NW
G = CHUNK // L

KROWS = 56

TI = 512
TB = 1024


def _pre_body(v_hbm, f_hbm, p_hbm, out_hbm, v_v, f_v, p_v, d_v):
    wid = lax.axis_index("s") * NC + lax.axis_index("c")
    base = wid * CHUNK
    pltpu.sync_copy(v_hbm, v_v)
    pltpu.sync_copy(f_hbm.at[pl.ds(base * 3, CHUNK * 3)], f_v)
    pltpu.sync_copy(p_hbm.at[pl.ds(base, CHUNK)], p_v)

    zf = jnp.zeros((L,), jnp.float32)
    onef = jnp.full((L,), 1.0, jnp.float32)
    negf = jnp.full((L,), -1.0, jnp.float32)

    for g in range(G):
        sl = pl.ds(g * L, L)
        fid3 = (lax.iota(jnp.int32, L) + g * L) * 3
        i0 = plsc.load_gather(f_v, [fid3]) * 3
        i1 = plsc.load_gather(f_v, [fid3 + 1]) * 3
        i2 = plsc.load_gather(f_v, [fid3 + 2]) * 3
        p0x = plsc.load_gather(v_v, [i0])
        p0y = plsc.load_gather(v_v, [i0 + 1])
        p0z = plsc.load_gather(v_v, [i0 + 2])
        p1x = plsc.load_gather(v_v, [i1])
        p1y = plsc.load_gather(v_v, [i1 + 1])
        p1z = plsc.load_gather(v_v, [i1 + 2])
        p2x = plsc.load_gather(v_v, [i2])
        p2y = plsc.load_gather(v_v, [i2 + 1])
        p2z = plsc.load_gather(v_v, [i2 + 2])

        e1x = p1x - p0x
        e1y = p1y - p0y
        e1z = p1z - p0z
        e2x = p2x - p0x
        e2y = p2y - p0y
        e2z = p2z - p0z
        nx = e1y * e2z - e1z * e2y
        ny = e1z * e2x - e1x * e2z
        nz = e1x * e2y - e1y * e2x
        ns = nx + ny + nz

        cx = (p0x + p1x + p2x) / 3.0
        cy = (p0y + p1y + p2y) / 3.0
        cz = (p0z + p1z + p2z) / 3.0
        sq = cx * cx + cy * cy + cz * cz
        pr = p_v[sl]

        d_v[0, sl] = nx
        d_v[1, sl] = ny
        d_v[2, sl] = nz
        d_v[3, sl] = p0x * ns
        d_v[4, sl] = p0y * ns
        d_v[5, sl] = p0z * ns
        d_v[8, sl] = p0x
        d_v[9, sl] = p1x
        d_v[10, sl] = p2x
        d_v[11, sl] = negf
        d_v[16, sl] = p0y
        d_v[17, sl] = p1y
        d_v[18, sl] = p2y
        d_v[20, sl] = negf
        d_v[24, sl] = p0z
        d_v[25, sl] = p1z
        d_v[26, sl] = p2z
        d_v[29, sl] = negf
        d_v[32, sl] = cx
        d_v[33, sl] = cy
        d_v[34, sl] = cz
        d_v[35, sl] = sq
        d_v[36, sl] = onef
        d_v[40, sl] = -2.0 * cx
        d_v[41, sl] = -2.0 * cy
        d_v[42, sl] = -2.0 * cz
        d_v[43, sl] = onef
        d_v[44, sl] = sq - 1.0
        d_v[48, sl] = pr
        for r in (6, 7, 12, 13, 14, 15, 19, 21, 22, 23,
                  27, 28, 30, 31, 37, 38, 39, 45, 46, 47):
            d_v[r, sl] = zf

    pltpu.sync_copy(d_v, out_hbm.at[:, pl.ds(base, CHUNK)])


@functools.cache
def _make_pre():
    return pl.kernel(
        _pre_body,
        out_type=jax.ShapeDtypeStruct((KROWS, F), jnp.float32),
        mesh=plsc.VectorSubcoreMesh(
            core_axis_name="c", subcore_axis_name="s",
            num_cores=NC, num_subcores=NS),
        scratch_types=[
            pltpu.VMEM((V * 3,), jnp.float32),
            pltpu.VMEM((CHUNK * 3,), jnp.int32),
            pltpu.VMEM((CHUNK,), jnp.float32),
            pltpu.VMEM((KROWS, CHUNK), jnp.float32),
        ],
        compiler_params=pltpu.CompilerParams(needs_layout_passes=False),
    )


def _pair_body(a_ref, b_ref, o_ref):
    a = a_ref[...]
    b = b_ref[...]
    dn = (((0,), (0,)), ((), ()))
    rhs3 = jnp.concatenate([b[8:16], b[16:24], b[24:32]], axis=1)
    s_all = lax.dot_general(a[0:8], rhs3, dn,
                            preferred_element_type=jnp.float32)
    s0 = s_all[:, 0:TB]
    s1 = s_all[:, TB:2 * TB]
    s2 = s_all[:, 2 * TB:3 * TB]
    h = lax.dot_general(a[32:40], b[40:48], dn,
                        preferred_element_type=jnp.float32)
    t = jnp.maximum(jnp.minimum(jnp.minimum(s0 * s2, s1 * s0), s2 * s1), h)
    ii = pl.program_id(0)
    bb = pl.program_id(1)
    ri = ii * TI + lax.broadcasted_iota(jnp.int32, (TI, 1), 0)
    ci = bb * TB + lax.broadcasted_iota(jnp.int32, (1, TB), 1)
    w = ((t < 0.0) & (ri != ci)).astype(jnp.float32)
    pr = a[48:49]
    row = lax.dot_general(pr, w, (((1,), (0,)), ((), ())),
                          preferred_element_type=jnp.float32)
    val = jnp.sum(row)

    @pl.when((ii == 0) & (bb == 0))
    def _init():
        o_ref[0, 0] = 0.0

    o_ref[0, 0] += val


_pair = pl.pallas_call(
    _pair_body,
    grid=(F // TI, F // TB),
    in_specs=[
        pl.BlockSpec((KROWS, TI), lambda i, b: (0, i)),
        pl.BlockSpec((KROWS, TB), lambda i, b: (0, b)),
    ],
    out_specs=pl.BlockSpec(memory_space=pltpu.SMEM),
    out_shape=jax.ShapeDtypeStruct((1, 1), jnp.float32),
    compiler_params=pltpu.CompilerParams(fuse_transposed_lhs_in_matmul=True),
)


def kernel(vertices, faces, face_probs):
    d = _make_pre()(vertices.reshape(-1), faces.reshape(-1), face_probs)
    tot = _pair(d, d)
    return tot[0, 0] / jnp.float32(F)


# speedup vs baseline: 737.0112x; 1.1379x over previous
import functools

import jax
import jax.numpy as jnp
from jax import lax
from jax.experimental import pallas as pl
from jax.experimental.pallas import tpu as pltpu
from jax.experimental.pallas import tpu_sc as plsc

V = 4096
F = 8192

NC = 2
NS = 16
NW = NC * NS
L = 16
CHUNK = F // NW
G = CHUNK // L

KROWS = 56

TI = 512
TB = 2048


def _pre_body(v_hbm, f_hbm, p_hbm, out_hbm, v_v, f_v, p_v, d_v):
    wid = lax.axis_index("s") * NC + lax.axis_index("c")
    base = wid * CHUNK
    pltpu.sync_copy(v_hbm, v_v)
    pltpu.sync_copy(f_hbm.at[pl.ds(base * 3, CHUNK * 3)], f_v)
    pltpu.sync_copy(p_hbm.at[pl.ds(base, CHUNK)], p_v)

    zf = jnp.zeros((L,), jnp.float32)
    onef = jnp.full((L,), 1.0, jnp.float32)
    negf = jnp.full((L,), -1.0, jnp.float32)

    for g in range(G):
        sl = pl.ds(g * L, L)
        fid3 = (lax.iota(jnp.int32, L) + g * L) * 3
        i0 = plsc.load_gather(f_v, [fid3]) * 3
        i1 = plsc.load_gather(f_v, [fid3 + 1]) * 3
        i2 = plsc.load_gather(f_v, [fid3 + 2]) * 3
        p0x = plsc.load_gather(v_v, [i0])
        p0y = plsc.load_gather(v_v, [i0 + 1])
        p0z = plsc.load_gather(v_v, [i0 + 2])
        p1x = plsc.load_gather(v_v, [i1])
        p1y = plsc.load_gather(v_v, [i1 + 1])
        p1z = plsc.load_gather(v_v, [i1 + 2])
        p2x = plsc.load_gather(v_v, [i2])
        p2y = plsc.load_gather(v_v, [i2 + 1])
        p2z = plsc.load_gather(v_v, [i2 + 2])

        e1x = p1x - p0x
        e1y = p1y - p0y
        e1z = p1z - p0z
        e2x = p2x - p0x
        e2y = p2y - p0y
        e2z = p2z - p0z
        nx = e1y * e2z - e1z * e2y
        ny = e1z * e2x - e1x * e2z
        nz = e1x * e2y - e1y * e2x
        ns = nx + ny + nz

        cx = (p0x + p1x + p2x) / 3.0
        cy = (p0y + p1y + p2y) / 3.0
        cz = (p0z + p1z + p2z) / 3.0
        sq = cx * cx + cy * cy + cz * cz
        pr = p_v[sl] * (1.0 / 8192.0)
        sx = nx * p0x + ny * p1x + nz * p2x - p0x * ns
        sy = nx * p0y + ny * p1y + nz * p2y - p0y * ns
        sz = nx * p0z + ny * p1z + nz * p2z - p0z * ns
        m3 = jnp.minimum(jnp.minimum(sx * sz, sy * sx), sz * sy)

        d_v[0, sl] = nx
        d_v[1, sl] = ny
        d_v[2, sl] = nz
        d_v[3, sl] = p0x * ns
        d_v[4, sl] = p0y * ns
        d_v[5, sl] = p0z * ns
        d_v[8, sl] = p0x
        d_v[9, sl] = p1x
        d_v[10, sl] = p2x
        d_v[11, sl] = negf
        d_v[16, sl] = p0y
        d_v[17, sl] = p1y
        d_v[18, sl] = p2y
        d_v[20, sl] = negf
        d_v[24, sl] = p0z
        d_v[25, sl] = p1z
        d_v[26, sl] = p2z
        d_v[29, sl] = negf
        d_v[32, sl] = cx
        d_v[33, sl] = cy
        d_v[34, sl] = cz
        d_v[35, sl] = sq
        d_v[36, sl] = onef
        d_v[40, sl] = -2.0 * cx
        d_v[41, sl] = -2.0 * cy
        d_v[42, sl] = -2.0 * cz
        d_v[43, sl] = onef
        d_v[44, sl] = sq - 1.0
        d_v[48, sl] = pr
        d_v[49, sl] = jnp.where(m3 < 0.0, pr, zf)
        for r in (6, 7, 12, 13, 14, 15, 19, 21, 22, 23,
                  27, 28, 30, 31, 37, 38, 39, 45, 46, 47):
            d_v[r, sl] = zf

    pltpu.sync_copy(d_v, out_hbm.at[:, pl.ds(base, CHUNK)])


@functools.cache
def _make_pre():
    return pl.kernel(
        _pre_body,
        out_type=jax.ShapeDtypeStruct((KROWS, F), jnp.float32),
        mesh=plsc.VectorSubcoreMesh(
            core_axis_name="c", subcore_axis_name="s",
            num_cores=NC, num_subcores=NS),
        scratch_types=[
            pltpu.VMEM((V * 3,), jnp.float32),
            pltpu.VMEM((CHUNK * 3,), jnp.int32),
            pltpu.VMEM((CHUNK,), jnp.float32),
            pltpu.VMEM((KROWS, CHUNK), jnp.float32),
        ],
        compiler_params=pltpu.CompilerParams(needs_layout_passes=False),
    )


def _pair_body(a_ref, b_ref, o_ref):
    a = a_ref[...]
    b = b_ref[...]
    dn = (((0,), (0,)), ((), ()))
    rhs3 = jnp.concatenate([b[8:16], b[16:24], b[24:32]], axis=1)
    s_all = lax.dot_general(a[0:8], rhs3, dn,
                            preferred_element_type=jnp.float32)
    s0 = s_all[:, 0:TB]
    s1 = s_all[:, TB:2 * TB]
    s2 = s_all[:, 2 * TB:3 * TB]
    h = lax.dot_general(a[32:40], b[40:48], dn,
                        preferred_element_type=jnp.float32)
    t = jnp.maximum(jnp.minimum(jnp.minimum(s0 * s2, s1 * s0), s2 * s1), h)
    ii = pl.program_id(0)
    bb = pl.program_id(1)
    w = (t < 0.0).astype(jnp.float32)
    pr = a[48:49]
    row = lax.dot_general(pr, w, (((1,), (0,)), ((), ())),
                          preferred_element_type=jnp.float32)
    val = jnp.sum(row) - jnp.where(bb == 0, jnp.sum(a[49:50]), 0.0)

    @pl.when((ii == 0) & (bb == 0))
    def _init():
        o_ref[0, 0] = 0.0

    o_ref[0, 0] += val


_pair = pl.pallas_call(
    _pair_body,
    grid=(F // TI, F // TB),
    in_specs=[
        pl.BlockSpec((KROWS, TI), lambda i, b: (0, i)),
        pl.BlockSpec((KROWS, TB), lambda i, b: (0, b)),
    ],
    out_specs=pl.BlockSpec(memory_space=pltpu.SMEM),
    out_shape=jax.ShapeDtypeStruct((1, 1), jnp.float32),
    compiler_params=pltpu.CompilerParams(fuse_transposed_lhs_in_matmul=True),
)


def kernel(vertices, faces, face_probs):
    d = _make_pre()(vertices.reshape(-1), faces.reshape(-1), face_probs)
    tot = _pair(d, d)
    return tot[0, 0]


# speedup vs baseline: 807.5747x; 1.0957x over previous
import functools

import jax
import jax.numpy as jnp
from jax import lax
from jax.experimental import pallas as pl
from jax.experimental.pallas import tpu as pltpu
from jax.experimental.pallas import tpu_sc as plsc

V = 4096
F = 8192

NC = 2
NS = 16
NW = NC * NS
L = 16
CHUNK = F // NW
G = CHUNK // L

KROWS = 56

TI = 512
TB = 2048


def _pre_body(v_hbm, f_hbm, p_hbm, out_hbm, v_v, f_v, p_v, d_v):
    wid = lax.axis_index("s") * NC + lax.axis_index("c")
    base = wid * CHUNK
    pltpu.sync_copy(v_hbm, v_v)
    pltpu.sync_copy(f_hbm.at[pl.ds(base * 3, CHUNK * 3)], f_v)
    pltpu.sync_copy(p_hbm.at[pl.ds(base, CHUNK)], p_v)

    zf = jnp.zeros((L,), jnp.float32)
    onef = jnp.full((L,), 1.0, jnp.float32)
    negf = jnp.full((L,), -1.0, jnp.float32)

    for g in range(G):
        sl = pl.ds(g * L, L)
        fid3 = (lax.iota(jnp.int32, L) + g * L) * 3
        i0 = plsc.load_gather(f_v, [fid3]) * 3
        i1 = plsc.load_gather(f_v, [fid3 + 1]) * 3
        i2 = plsc.load_gather(f_v, [fid3 + 2]) * 3
        p0x = plsc.load_gather(v_v, [i0])
        p0y = plsc.load_gather(v_v, [i0 + 1])
        p0z = plsc.load_gather(v_v, [i0 + 2])
        p1x = plsc.load_gather(v_v, [i1])
        p1y = plsc.load_gather(v_v, [i1 + 1])
        p1z = plsc.load_gather(v_v, [i1 + 2])
        p2x = plsc.load_gather(v_v, [i2])
        p2y = plsc.load_gather(v_v, [i2 + 1])
        p2z = plsc.load_gather(v_v, [i2 + 2])

        e1x = p1x - p0x
        e1y = p1y - p0y
        e1z = p1z - p0z
        e2x = p2x - p0x
        e2y = p2y - p0y
        e2z = p2z - p0z
        nx = e1y * e2z - e1z * e2y
        ny = e1z * e2x - e1x * e2z
        nz = e1x * e2y - e1y * e2x
        ns = nx + ny + nz

        cx = (p0x + p1x + p2x) / 3.0
        cy = (p0y + p1y + p2y) / 3.0
        cz = (p0z + p1z + p2z) / 3.0
        sq = cx * cx + cy * cy + cz * cz
        pr = p_v[sl] * (1.0 / 8192.0)
        sx = nx * p0x + ny * p1x + nz * p2x - p0x * ns
        sy = nx * p0y + ny * p1y + nz * p2y - p0y * ns
        sz = nx * p0z + ny * p1z + nz * p2z - p0z * ns
        m3 = jnp.minimum(jnp.minimum(sx * sz, sy * sx), sz * sy)

        d_v[0, sl] = nx
        d_v[1, sl] = ny
        d_v[2, sl] = nz
        d_v[3, sl] = p0x * ns
        d_v[4, sl] = p0y * ns
        d_v[5, sl] = p0z * ns
        d_v[8, sl] = p0x
        d_v[9, sl] = p1x
        d_v[10, sl] = p2x
        d_v[11, sl] = negf
        d_v[16, sl] = p0y
        d_v[17, sl] = p1y
        d_v[18, sl] = p2y
        d_v[20, sl] = negf
        d_v[24, sl] = p0z
        d_v[25, sl] = p1z
        d_v[26, sl] = p2z
        d_v[29, sl] = negf
        d_v[32, sl] = cx
        d_v[33, sl] = cy
        d_v[34, sl] = cz
        d_v[35, sl] = sq
        d_v[36, sl] = onef
        d_v[40, sl] = -2.0 * cx
        d_v[41, sl] = -2.0 * cy
        d_v[42, sl] = -2.0 * cz
        d_v[43, sl] = onef
        d_v[44, sl] = sq - 1.0
        d_v[48, sl] = pr
        d_v[49, sl] = jnp.where(m3 < 0.0, pr, zf)
        for r in (6, 7, 12, 13, 14, 15, 19, 21, 22, 23,
                  27, 28, 30, 31, 37, 38, 39, 45, 46, 47):
            d_v[r, sl] = zf

    pltpu.sync_copy(d_v, out_hbm.at[:, pl.ds(base, CHUNK)])


@functools.cache
def _make_pre():
    return pl.kernel(
        _pre_body,
        out_type=jax.ShapeDtypeStruct((KROWS, F), jnp.float32),
        mesh=plsc.VectorSubcoreMesh(
            core_axis_name="c", subcore_axis_name="s",
            num_cores=NC, num_subcores=NS),
        scratch_types=[
            pltpu.VMEM((V * 3,), jnp.float32),
            pltpu.VMEM((CHUNK * 3,), jnp.int32),
            pltpu.VMEM((CHUNK,), jnp.float32),
            pltpu.VMEM((KROWS, CHUNK), jnp.float32),
        ],
        compiler_params=pltpu.CompilerParams(needs_layout_passes=False),
    )


def _pair_body(a_ref, b_ref, o_ref, acc_ref):
    a = a_ref[...]
    b = b_ref[...]
    dn = (((0,), (0,)), ((), ()))
    rhs3 = jnp.concatenate([b[8:16], b[16:24], b[24:32]], axis=1)
    s_all = lax.dot_general(a[0:8], rhs3, dn,
                            preferred_element_type=jnp.float32)
    s0 = s_all[:, 0:TB]
    s1 = s_all[:, TB:2 * TB]
    s2 = s_all[:, 2 * TB:3 * TB]
    h = lax.dot_general(a[32:40], b[40:48], dn,
                        preferred_element_type=jnp.float32)
    t = jnp.maximum(jnp.minimum(jnp.minimum(s0 * s2, s1 * s0), s2 * s1), h)
    ii = pl.program_id(0)
    bb = pl.program_id(1)
    w = (t < 0.0).astype(jnp.float32)
    wsum = jnp.sum(w, axis=1, keepdims=True)

    @pl.when(bb == 0)
    def _acc_init():
        acc_ref[...] = wsum

    @pl.when(bb > 0)
    def _acc_add():
        acc_ref[...] += wsum

    @pl.when((ii == 0) & (bb == 0))
    def _init():
        o_ref[0, 0] = 0.0

    @pl.when(bb == F // TB - 1)
    def _finalize():
        pr = a[48:49]
        row = lax.dot_general(pr, acc_ref[...], (((1,), (0,)), ((), ())),
                              preferred_element_type=jnp.float32)
        o_ref[0, 0] += row[0, 0] - jnp.sum(a[49:50])


_pair = pl.pallas_call(
    _pair_body,
    grid=(F // TI, F // TB),
    in_specs=[
        pl.BlockSpec((KROWS, TI), lambda i, b: (0, i)),
        pl.BlockSpec((KROWS, TB), lambda i, b: (0, b)),
    ],
    out_specs=pl.BlockSpec(memory_space=pltpu.SMEM),
    out_shape=jax.ShapeDtypeStruct((1, 1), jnp.float32),
    scratch_shapes=[pltpu.VMEM((TI, 1), jnp.float32)],
    compiler_params=pltpu.CompilerParams(fuse_transposed_lhs_in_matmul=True),
)


def kernel(vertices, faces, face_probs):
    d = _make_pre()(vertices.reshape(-1), faces.reshape(-1), face_probs)
    tot = _pair(d, d)
    return tot[0, 0]


# speedup vs baseline: 819.2236x; 1.0144x over previous
import functools

import jax
import jax.numpy as jnp
from jax import lax
from jax.experimental import pallas as pl
from jax.experimental.pallas import tpu as pltpu
from jax.experimental.pallas import tpu_sc as plsc

V = 4096
F = 8192

NC = 2
NS = 16
NW = NC * NS
L = 16
CHUNK = F // NW
G = CHUNK // L

KROWS = 56

TI = 512
TB = 4096


def _pre_body(v_hbm, f_hbm, p_hbm, out_hbm, v_v, f_v, p_v, d_v):
    wid = lax.axis_index("s") * NC + lax.axis_index("c")
    base = wid * CHUNK
    pltpu.sync_copy(v_hbm, v_v)
    pltpu.sync_copy(f_hbm.at[pl.ds(base * 3, CHUNK * 3)], f_v)
    pltpu.sync_copy(p_hbm.at[pl.ds(base, CHUNK)], p_v)

    zf = jnp.zeros((L,), jnp.float32)
    onef = jnp.full((L,), 1.0, jnp.float32)
    negf = jnp.full((L,), -1.0, jnp.float32)

    for g in range(G):
        sl = pl.ds(g * L, L)
        fid3 = (lax.iota(jnp.int32, L) + g * L) * 3
        i0 = plsc.load_gather(f_v, [fid3]) * 3
        i1 = plsc.load_gather(f_v, [fid3 + 1]) * 3
        i2 = plsc.load_gather(f_v, [fid3 + 2]) * 3
        p0x = plsc.load_gather(v_v, [i0])
        p0y = plsc.load_gather(v_v, [i0 + 1])
        p0z = plsc.load_gather(v_v, [i0 + 2])
        p1x = plsc.load_gather(v_v, [i1])
        p1y = plsc.load_gather(v_v, [i1 + 1])
        p1z = plsc.load_gather(v_v, [i1 + 2])
        p2x = plsc.load_gather(v_v, [i2])
        p2y = plsc.load_gather(v_v, [i2 + 1])
        p2z = plsc.load_gather(v_v, [i2 + 2])

        e1x = p1x - p0x
        e1y = p1y - p0y
        e1z = p1z - p0z
        e2x = p2x - p0x
        e2y = p2y - p0y
        e2z = p2z - p0z
        nx = e1y * e2z - e1z * e2y
        ny = e1z * e2x - e1x * e2z
        nz = e1x * e2y - e1y * e2x
        ns = nx + ny + nz

        cx = (p0x + p1x + p2x) / 3.0
        cy = (p0y + p1y + p2y) / 3.0
        cz = (p0z + p1z + p2z) / 3.0
        sq = cx * cx + cy * cy + cz * cz
        pr = p_v[sl] * (1.0 / 8192.0)
        sx = nx * p0x + ny * p1x + nz * p2x - p0x * ns
        sy = nx * p0y + ny * p1y + nz * p2y - p0y * ns
        sz = nx * p0z + ny * p1z + nz * p2z - p0z * ns
        m3 = jnp.minimum(jnp.minimum(sx * sz, sy * sx), sz * sy)

        d_v[0, sl] = nx
        d_v[1, sl] = ny
        d_v[2, sl] = nz
        d_v[3, sl] = p0x * ns
        d_v[4, sl] = p0y * ns
        d_v[5, sl] = p0z * ns
        d_v[8, sl] = p0x
        d_v[9, sl] = p1x
        d_v[10, sl] = p2x
        d_v[11, sl] = negf
        d_v[16, sl] = p0y
        d_v[17, sl] = p1y
        d_v[18, sl] = p2y
        d_v[20, sl] = negf
        d_v[24, sl] = p0z
        d_v[25, sl] = p1z
        d_v[26, sl] = p2z
        d_v[29, sl] = negf
        d_v[32, sl] = cx
        d_v[33, sl] = cy
        d_v[34, sl] = cz
        d_v[35, sl] = sq
        d_v[36, sl] = onef
        d_v[40, sl] = -2.0 * cx
        d_v[41, sl] = -2.0 * cy
        d_v[42, sl] = -2.0 * cz
        d_v[43, sl] = onef
        d_v[44, sl] = sq - 1.0
        d_v[48, sl] = pr
        d_v[49, sl] = jnp.where(m3 < 0.0, pr, zf)
        for r in (6, 7, 12, 13, 14, 15, 19, 21, 22, 23,
                  27, 28, 30, 31, 37, 38, 39, 45, 46, 47):
            d_v[r, sl] = zf

    pltpu.sync_copy(d_v, out_hbm.at[:, pl.ds(base, CHUNK)])


@functools.cache
def _make_pre():
    return pl.kernel(
        _pre_body,
        out_type=jax.ShapeDtypeStruct((KROWS, F), jnp.float32),
        mesh=plsc.VectorSubcoreMesh(
            core_axis_name="c", subcore_axis_name="s",
            num_cores=NC, num_subcores=NS),
        scratch_types=[
            pltpu.VMEM((V * 3,), jnp.float32),
            pltpu.VMEM((CHUNK * 3,), jnp.int32),
            pltpu.VMEM((CHUNK,), jnp.float32),
            pltpu.VMEM((KROWS, CHUNK), jnp.float32),
        ],
        compiler_params=pltpu.CompilerParams(needs_layout_passes=False),
    )


def _pair_body(a_ref, b_ref, o_ref, acc_ref):
    a = a_ref[...]
    b = b_ref[...]
    dn = (((0,), (0,)), ((), ()))
    rhs3 = jnp.concatenate([b[8:16], b[16:24], b[24:32]], axis=1)
    s_all = lax.dot_general(a[0:8], rhs3, dn,
                            preferred_element_type=jnp.float32)
    s0 = s_all[:, 0:TB]
    s1 = s_all[:, TB:2 * TB]
    s2 = s_all[:, 2 * TB:3 * TB]
    h = lax.dot_general(a[32:40], b[40:48], dn,
                        preferred_element_type=jnp.float32)
    t = jnp.maximum(jnp.minimum(jnp.minimum(s0 * s2, s1 * s0), s2 * s1), h)
    ii = pl.program_id(0)
    bb = pl.program_id(1)
    w = (t < 0.0).astype(jnp.float32)
    wsum = jnp.sum(w, axis=1, keepdims=True)

    @pl.when(bb == 0)
    def _acc_init():
        acc_ref[...] = wsum

    @pl.when(bb > 0)
    def _acc_add():
        acc_ref[...] += wsum

    @pl.when((ii == 0) & (bb == 0))
    def _init():
        o_ref[0, 0] = 0.0

    @pl.when(bb == F // TB - 1)
    def _finalize():
        pr = a[48:49]
        row = lax.dot_general(pr, acc_ref[...], (((1,), (0,)), ((), ())),
                              preferred_element_type=jnp.float32)
        o_ref[0, 0] += row[0, 0] - jnp.sum(a[49:50])


_pair = pl.pallas_call(
    _pair_body,
    grid=(F // TI, F // TB),
    in_specs=[
        pl.BlockSpec((KROWS, TI), lambda i, b: (0, i)),
        pl.BlockSpec((KROWS, TB), lambda i, b: (0, b)),
    ],
    out_specs=pl.BlockSpec(memory_space=pltpu.SMEM),
    out_shape=jax.ShapeDtypeStruct((1, 1), jnp.float32),
    scratch_shapes=[pltpu.VMEM((TI, 1), jnp.float32)],
    compiler_params=pltpu.CompilerParams(fuse_transposed_lhs_in_matmul=True),
)


def kernel(vertices, faces, face_probs):
    d = _make_pre()(vertices.reshape(-1), faces.reshape(-1), face_probs)
    tot = _pair(d, d)
    return tot[0, 0]
